# baseline (speedup 1.0000x reference)
import functools

import jax
import jax.numpy as jnp
from jax import lax
from jax.experimental import pallas as pl
from jax.experimental.pallas import tpu as pltpu
from jax.experimental.pallas import tpu_sc as plsc

_NC = 2
_NS = 16
_NW = _NC * _NS
_LANES = 16


def _ff_body(x_ref, w1_ref, b1_ref, w2_ref, o_ref, xb_ref):
    xf = x_ref[...]
    x = xf.astype(jnp.bfloat16)
    half = xf.shape[1] // 2
    u = lax.bitcast_convert_type(xf, jnp.uint32)
    r = (u + 0x7FFF + ((u >> 16) & 1)) >> 16
    word = r[:, :half] | (r[:, half:] << 16)
    xb_ref[...] = lax.bitcast_convert_type(word, jnp.int32)
    w1 = w1_ref[...].astype(jnp.bfloat16)
    h = jnp.dot(x, w1, preferred_element_type=jnp.float32)
    h = jnp.maximum(h + b1_ref[...][None, :], 0.0)
    s = jnp.dot(h.astype(jnp.bfloat16), w2_ref[...].astype(jnp.bfloat16),
                preferred_element_type=jnp.float32)
    o_ref[...] = s[:, 0]


def _ff_scores(mv, W1, b1, W2):
    n, d = mv.shape
    hdim = W1.shape[1]
    blk = 2048
    grid = n // blk
    return pl.pallas_call(
        _ff_body,
        grid=(grid,),
        in_specs=[
            pl.BlockSpec((blk, d), lambda i: (i, 0)),
            pl.BlockSpec((d, hdim), lambda i: (0, 0)),
            pl.BlockSpec((hdim,), lambda i: (0,)),
            pl.BlockSpec((hdim, 1), lambda i: (0, 0)),
        ],
        out_specs=[
            pl.BlockSpec((blk,), lambda i: (i,)),
            pl.BlockSpec((blk, d // 2), lambda i: (i, 0)),
        ],
        out_shape=[
            jax.ShapeDtypeStruct((n,), jnp.float32),
            jax.ShapeDtypeStruct((n, d // 2), jnp.int32),
        ],
    )(mv, W1, b1, W2)


def _sc_attention(scores, idx, lengths, mv, D):
    C, L = idx.shape
    N, Dw = mv.shape
    cpt = C // _NW
    npair = D // (2 * _LANES)

    mesh = plsc.VectorSubcoreMesh(core_axis_name="c", subcore_axis_name="s")

    grp = 8

    @functools.partial(
        pl.kernel,
        mesh=mesh,
        compiler_params=pltpu.CompilerParams(needs_layout_passes=False),
        out_type=jax.ShapeDtypeStruct((C, D), jnp.float32),
        scratch_types=[
            pltpu.VMEM((N,), jnp.float32),
            pltpu.VMEM((cpt, L), jnp.int32),
            pltpu.VMEM((cpt + _LANES,), jnp.int32),
            [pltpu.VMEM((L, Dw), jnp.int32)] * 4,
            pltpu.VMEM((L + _LANES,), jnp.float32),
            [pltpu.VMEM((4, D), jnp.float32)] * 2,
            [pltpu.SemaphoreType.DMA] * 4,
            [pltpu.SemaphoreType.DMA] * 2,
        ],
    )
    def k(scores_hbm, idx_hbm, len_hbm, mv_hbm, out_hbm,
          scores_v, idx_v, len_v, rows_bufs, probs_v, acc_bufs,
          gsems, osems):
        wid = lax.axis_index("s") * _NC + lax.axis_index("c")
        base = wid * cpt
        pltpu.sync_copy(scores_hbm, scores_v)
        pltpu.sync_copy(idx_hbm.at[pl.ds(base, cpt)], idx_v)
        pltpu.sync_copy(len_hbm.at[pl.ds(base, cpt)], len_v.at[pl.ds(0, cpt)])

        lane = lax.iota(jnp.int32, _LANES)

        def nchunks(c):
            ln = len_v[pl.ds(c, _LANES)][0]
            return (ln + (grp - 1)) // grp

        def fire(c, rows_ref, sem):
            k = nchunks(c)
            for kk in range(1, L // grp + 1):
                @pl.when(k == kk)
                def _():
                    pltpu.async_copy(
                        mv_hbm.at[idx_v.at[c, pl.ds(0, kk * grp)]],
                        rows_ref.at[pl.ds(0, kk * grp)], sem)

        def wait_rows(c, rows_ref, sem):
            k = nchunks(c)
            for kk in range(1, L // grp + 1):
                @pl.when(k == kk)
                def _():
                    pltpu.make_async_copy(
                        mv_hbm.at[idx_v.at[c, pl.ds(0, kk * grp)]],
                        rows_ref.at[pl.ds(0, kk * grp)], sem).wait()

        def process(c, rows_ref, gsem, acc_ref, arow):
            i0 = idx_v[c, pl.ds(0, _LANES)]
            i1 = idx_v[c, pl.ds(_LANES, _LANES)]
            s0 = plsc.load_gather(scores_v, [i0])
            s1 = plsc.load_gather(scores_v, [i1])
            ln = len_v[pl.ds(c, _LANES)][0]
            lnv = jnp.full((_LANES,), ln, dtype=jnp.int32)
            m0 = lane < lnv
            m1 = (lane + _LANES) < lnv
            s0 = jnp.where(m0, s0, s0 - 1e38)
            s1 = jnp.where(m1, s1, s1 - 1e38)
            mx = jnp.max(jnp.maximum(s0, s1))
            e0 = jnp.exp(s0 - mx)
            e1 = jnp.exp(s1 - mx)
            denom = jnp.sum(e0 + e1)
            probs_v[pl.ds(0, _LANES)] = e0 / denom
            probs_v[pl.ds(_LANES, _LANES)] = e1 / denom

            wait_rows(c, rows_ref, gsem)

            def fma(l, accs):
                pvec = jnp.full((_LANES,), probs_v[pl.ds(l, _LANES)][0],
                                dtype=jnp.float32)
                new = []
                for j in range(npair):
                    chunk = plsc.bitcast(
                        rows_ref[l, pl.ds(j * _LANES, _LANES)], jnp.bfloat16)
                    a, b = plsc.unpack(chunk, format=plsc.PackFormat.INTERLEAVED)
                    new.append(accs[2 * j] + pvec * a)
                    new.append(accs[2 * j + 1] + pvec * b)
                return tuple(new)

            zero = jnp.zeros((_LANES,), jnp.float32)
            accs = lax.fori_loop(0, ln, fma,
                                 tuple(zero for _ in range(2 * npair)))

            for j in range(npair):
                acc_ref[arow, pl.ds(j * _LANES, _LANES)] = accs[2 * j]
                acc_ref[arow, pl.ds(D // 2 + j * _LANES, _LANES)] = \
                    accs[2 * j + 1]

        nbuf = 4
        nquads = cpt // nbuf
        for kk in range(nbuf):
            fire(kk, rows_bufs[kk], gsems[kk])

        def quad_body(q, acc_ref, osem):
            @pl.when(q >= 2)
            def _():
                pltpu.make_async_copy(acc_ref, out_hbm.at[pl.ds(base, nbuf)],
                                      osem).wait()
            for kk in range(nbuf):
                c = nbuf * q + kk
                process(c, rows_bufs[kk], gsems[kk], acc_ref, kk)
                @pl.when(q + 1 < nquads)
                def _():
                    fire(c + nbuf, rows_bufs[kk], gsems[kk])
            pltpu.async_copy(acc_ref,
                             out_hbm.at[pl.ds(base + nbuf * q, nbuf)], osem)

        def quad_pair(h, carry):
            quad_body(2 * h, acc_bufs[0], osems[0])
            quad_body(2 * h + 1, acc_bufs[1], osems[1])
            return carry

        lax.fori_loop(0, nquads // 2, quad_pair, 0)
        for kk in range(2):
            pltpu.make_async_copy(acc_bufs[kk], out_hbm.at[pl.ds(base, nbuf)],
                                  osems[kk]).wait()

    return k(scores, idx, lengths, mv)


def kernel(mention_vectors, concept_indices, concept_lengths, W1, b1, W2, b2):
    num_batch, m, d = mention_vectors.shape
    mv = mention_vectors.reshape(-1, d)
    scores, mv_packed = _ff_scores(mv, W1, b1, W2)
    out = _sc_attention(scores, concept_indices, concept_lengths, mv_packed, d)
    return out.reshape(num_batch, -1, d)


# speedup vs baseline: 1.3101x; 1.3101x over previous
import functools

import jax
import jax.numpy as jnp
from jax import lax
from jax.experimental import pallas as pl
from jax.experimental.pallas import tpu as pltpu
from jax.experimental.pallas import tpu_sc as plsc

_NC = 2
_NS = 16
_NW = _NC * _NS
_LANES = 16


def _ff_body(x_ref, w1_ref, b1_ref, w2_ref, o_ref, xb_ref):
    xf = x_ref[...]
    x = xf.astype(jnp.bfloat16)
    half = xf.shape[1] // 2
    u = lax.bitcast_convert_type(xf, jnp.uint32)
    r = (u + 0x7FFF + ((u >> 16) & 1)) >> 16
    word = r[:, :half] | (r[:, half:] << 16)
    xb_ref[...] = lax.bitcast_convert_type(word, jnp.int32)
    w1 = w1_ref[...].astype(jnp.bfloat16)
    h = jnp.dot(x, w1, preferred_element_type=jnp.float32)
    h = jnp.maximum(h + b1_ref[...][None, :], 0.0)
    s = jnp.dot(h.astype(jnp.bfloat16), w2_ref[...].astype(jnp.bfloat16),
                preferred_element_type=jnp.float32)
    o_ref[...] = s[:, 0]


def _ff_scores(mv, W1, b1, W2):
    n, d = mv.shape
    hdim = W1.shape[1]
    blk = 2048
    grid = n // blk
    return pl.pallas_call(
        _ff_body,
        grid=(grid,),
        in_specs=[
            pl.BlockSpec((blk, d), lambda i: (i, 0)),
            pl.BlockSpec((d, hdim), lambda i: (0, 0)),
            pl.BlockSpec((hdim,), lambda i: (0,)),
            pl.BlockSpec((hdim, 1), lambda i: (0, 0)),
        ],
        out_specs=[
            pl.BlockSpec((blk,), lambda i: (i,)),
            pl.BlockSpec((blk, d // 2), lambda i: (i, 0)),
        ],
        out_shape=[
            jax.ShapeDtypeStruct((n,), jnp.float32),
            jax.ShapeDtypeStruct((n, d // 2), jnp.int32),
        ],
    )(mv, W1, b1, W2)


def _sc_attention(scores, idx, lengths, mv, D):
    C, L = idx.shape
    N, Dw = mv.shape
    cpt = C // _NW
    npair = D // (2 * _LANES)

    mesh = plsc.VectorSubcoreMesh(core_axis_name="c", subcore_axis_name="s")

    grp = 8

    @functools.partial(
        pl.kernel,
        mesh=mesh,
        compiler_params=pltpu.CompilerParams(needs_layout_passes=False),
        out_type=jax.ShapeDtypeStruct((C, D), jnp.float32),
        scratch_types=[
            pltpu.VMEM((N,), jnp.float32),
            pltpu.VMEM((cpt, L), jnp.int32),
            pltpu.VMEM((cpt + _LANES,), jnp.int32),
            [pltpu.VMEM((L, Dw), jnp.int32)] * 4,
            pltpu.VMEM((L + _LANES,), jnp.float32),
            [pltpu.VMEM((D,), jnp.float32)] * 4,
            [pltpu.SemaphoreType.DMA] * 4,
            [pltpu.SemaphoreType.DMA] * 4,
        ],
    )
    def k(scores_hbm, idx_hbm, len_hbm, mv_hbm, out_hbm,
          scores_v, idx_v, len_v, rows_bufs, probs_v, acc_bufs,
          gsems, osems):
        wid = lax.axis_index("s") * _NC + lax.axis_index("c")
        base = wid * cpt
        pltpu.sync_copy(scores_hbm, scores_v)
        pltpu.sync_copy(idx_hbm.at[pl.ds(base, cpt)], idx_v)
        pltpu.sync_copy(len_hbm.at[pl.ds(base, cpt)], len_v.at[pl.ds(0, cpt)])

        lane = lax.iota(jnp.int32, _LANES)

        def nchunks(c):
            ln = len_v[pl.ds(c, _LANES)][0]
            return (ln + (grp - 1)) // grp

        def fire(c, rows_ref, sem):
            k = nchunks(c)
            for kk in range(1, L // grp + 1):
                @pl.when(k == kk)
                def _():
                    pltpu.async_copy(
                        mv_hbm.at[idx_v.at[c, pl.ds(0, kk * grp)]],
                        rows_ref.at[pl.ds(0, kk * grp)], sem)

        def wait_rows(c, rows_ref, sem):
            k = nchunks(c)
            for kk in range(1, L // grp + 1):
                @pl.when(k == kk)
                def _():
                    pltpu.make_async_copy(
                        mv_hbm.at[idx_v.at[c, pl.ds(0, kk * grp)]],
                        rows_ref.at[pl.ds(0, kk * grp)], sem).wait()

        def process(c, rows_ref, gsem, acc_ref, osem, wait_out):
            i0 = idx_v[c, pl.ds(0, _LANES)]
            i1 = idx_v[c, pl.ds(_LANES, _LANES)]
            s0 = plsc.load_gather(scores_v, [i0])
            s1 = plsc.load_gather(scores_v, [i1])
            ln = len_v[pl.ds(c, _LANES)][0]
            lnv = jnp.full((_LANES,), ln, dtype=jnp.int32)
            m0 = lane < lnv
            m1 = (lane + _LANES) < lnv
            s0 = jnp.where(m0, s0, s0 - 1e38)
            s1 = jnp.where(m1, s1, s1 - 1e38)
            mx = jnp.max(jnp.maximum(s0, s1))
            e0 = jnp.exp(s0 - mx)
            e1 = jnp.exp(s1 - mx)
            denom = jnp.sum(e0 + e1)
            probs_v[pl.ds(0, _LANES)] = e0 / denom
            probs_v[pl.ds(_LANES, _LANES)] = e1 / denom

            wait_rows(c, rows_ref, gsem)

            def fma(l, accs):
                pvec = jnp.full((_LANES,), probs_v[pl.ds(l, _LANES)][0],
                                dtype=jnp.float32)
                new = []
                for j in range(npair):
                    chunk = plsc.bitcast(
                        rows_ref[l, pl.ds(j * _LANES, _LANES)], jnp.bfloat16)
                    a, b = plsc.unpack(chunk, format=plsc.PackFormat.INTERLEAVED)
                    new.append(accs[2 * j] + pvec * a)
                    new.append(accs[2 * j + 1] + pvec * b)
                return tuple(new)

            zero = jnp.zeros((_LANES,), jnp.float32)
            accs = lax.fori_loop(0, ln, fma,
                                 tuple(zero for _ in range(2 * npair)))

            @pl.when(wait_out)
            def _():
                pltpu.make_async_copy(acc_ref, out_hbm.at[base], osem).wait()
            for j in range(npair):
                acc_ref[pl.ds(j * _LANES, _LANES)] = accs[2 * j]
                acc_ref[pl.ds(D // 2 + j * _LANES, _LANES)] = accs[2 * j + 1]
            pltpu.async_copy(acc_ref, out_hbm.at[base + c], osem)

        nbuf = 4
        nquads = cpt // nbuf
        for kk in range(nbuf):
            fire(kk, rows_bufs[kk], gsems[kk])

        def quad(q, carry):
            for kk in range(nbuf):
                c = nbuf * q + kk
                process(c, rows_bufs[kk], gsems[kk], acc_bufs[kk], osems[kk],
                        q > 0)
                @pl.when(q + 1 < nquads)
                def _():
                    fire(c + nbuf, rows_bufs[kk], gsems[kk])
            return carry

        lax.fori_loop(0, nquads, quad, 0)
        for kk in range(nbuf):
            pltpu.make_async_copy(acc_bufs[kk], out_hbm.at[base],
                                  osems[kk]).wait()

    return k(scores, idx, lengths, mv)


def kernel(mention_vectors, concept_indices, concept_lengths, W1, b1, W2, b2):
    num_batch, m, d = mention_vectors.shape
    mv = mention_vectors.reshape(-1, d)
    scores, mv_packed = _ff_scores(mv, W1, b1, W2)
    out = _sc_attention(scores, concept_indices, concept_lengths, mv_packed, d)
    return out.reshape(num_batch, -1, d)
